# baseline (speedup 1.0000x reference)
import functools

import jax
import jax.numpy as jnp
from jax import lax
from jax.experimental import pallas as pl
from jax.experimental.pallas import tpu as pltpu
from jax.experimental.pallas import tpu_sc as plsc

D = 64
NROW = 1000
BATCH = 1024
BCH = 128
L = 16
NC, NS = 2, 16
NW = NC * NS


@functools.lru_cache(maxsize=None)
def _emb_kernel(P, S, W):
    NCH = BATCH // BCH
    npos = P // NW
    assert npos % 2 == 1 and npos >= 5 and P == S * W
    mesh = plsc.VectorSubcoreMesh(core_axis_name="c", subcore_axis_name="s")

    @functools.partial(
        pl.kernel,
        mesh=mesh,
        out_type=(
            jax.ShapeDtypeStruct((P, D // 8, NCH, 8, BCH), jnp.float32),
            jax.ShapeDtypeStruct((S, W // 8, NCH, 8, BCH), jnp.float32),
        ),
        scratch_types=[
            pltpu.VMEM((NROW * D,), jnp.float32),
            pltpu.VMEM((2, BATCH), jnp.int32),
            pltpu.VMEM((2, D // 8, 8, BCH), jnp.float32),
            pltpu.VMEM((2, BCH), jnp.float32),
            pltpu.SemaphoreType.DMA((2,)),
            pltpu.SemaphoreType.DMA((2,)),
            pltpu.SemaphoreType.DMA((2,)),
        ],
        compiler_params=pltpu.CompilerParams(needs_layout_passes=False),
    )
    def k(idx_hbm, tab_hbm, out_hbm, msk_hbm,
          table_v, idx_v, out_v, mask_v, s_idx, s_emb, s_msk):
        wid = lax.axis_index("s") * NC + lax.axis_index("c")
        p0 = wid * npos
        pltpu.sync_copy(tab_hbm, table_v)

        def idx_copy(p, b):
            return pltpu.make_async_copy(
                idx_hbm.at[p], idx_v.at[b], s_idx.at[b])

        def emb_copy(p, c, eb):
            return pltpu.make_async_copy(
                out_v.at[eb], out_hbm.at[p, :, c], s_emb.at[eb])

        def msk_copy(p, c, eb):
            s_i = p // W
            w_i = p % W
            return pltpu.make_async_copy(
                mask_v.at[eb], msk_hbm.at[s_i, w_i // 8, c, w_i % 8],
                s_msk.at[eb])

        idx_copy(p0, 0).start()

        def chunk(p, c, eb, reclaim, b):
            if reclaim:
                emb_copy(p, c, eb).wait()
                msk_copy(p, c, eb).wait()

            def vbody(v, carry):
                idx16 = idx_v[b, pl.ds(c * BCH + v * L, L)]
                base16 = idx16 * D
                mask_v[eb, pl.ds(v * L, L)] = jnp.where(
                    idx16 != 0, jnp.float32(1.0), jnp.float32(0.0))
                for d in range(D):
                    val = plsc.load_gather(table_v, [base16 + d])
                    out_v[eb, d // 8, d % 8, pl.ds(v * L, L)] = val
                return carry

            lax.fori_loop(0, BCH // L, vbody, 0)
            emb_copy(p, c, eb).start()
            msk_copy(p, c, eb).start()

        def do_pos(p, b, prefetch, first):
            idx_copy(p, b).wait()
            if prefetch:
                idx_copy(p + 1, 1 - b).start()
            if first:
                chunk(p, 0, 0, False, b)
                chunk(p, 1, 1, False, b)
                k0 = 1
            else:
                k0 = 0

            def cpair(kk, carry):
                c = 2 * kk
                chunk(p, c, 0, True, b)
                chunk(p, c + 1, 1, True, b)
                return carry

            lax.fori_loop(k0, NCH // 2, cpair, 0)

        do_pos(p0, 0, prefetch=True, first=True)
        do_pos(p0 + 1, 1, prefetch=True, first=False)

        def pbody(i, carry):
            p = p0 + 2 + 2 * i
            do_pos(p, 0, prefetch=True, first=False)
            do_pos(p + 1, 1, prefetch=True, first=False)
            return carry

        lax.fori_loop(0, (npos - 3) // 2, pbody, 0)
        do_pos(p0 + npos - 1, 0, prefetch=False, first=False)

        for eb in range(2):
            emb_copy(p0, 0, eb).wait()
            msk_copy(p0, 0, eb).wait()

    return k


def kernel(x, table):
    b, s, w = x.shape
    P = s * w
    idx = x.transpose(1, 2, 0).reshape(P, b).astype(jnp.int32)
    tab = table.astype(jnp.float32).reshape(-1)
    out5, mask5 = _emb_kernel(P, s, w)(idx, tab)
    x_emb = out5.transpose(2, 4, 0, 1, 3).reshape(b, s, w, D)
    mask = mask5.transpose(2, 4, 0, 1, 3).reshape(b, s, w)
    return (x_emb, mask)
